# baseline (speedup 1.0000x reference)
import functools

import jax
import jax.numpy as jnp
from jax import lax
from jax.experimental import pallas as pl
from jax.experimental.pallas import tpu as pltpu
from jax.experimental.pallas import tpu_sc as plsc

R = 100000
P = 14
ACD = 32
ATD = 16
RCD = 32
RID = 32
OUT = 256
A = R * P

BLK = 2048
NBLK = 49
RPAD = BLK * NBLK
APAD = RPAD * P
LW = 128
AROWS = APAD // LW
RROWS = RPAD // LW
NC, NS = 2, 16
NW = NC * NS
A_ROWS_PER_W = AROWS // NW
SUP = 7
A_BATCHES = A_ROWS_PER_W // SUP
R_ROWS_PER_HALF = RROWS // 16
R_BATCHES = R_ROWS_PER_HALF // SUP

@functools.cache
def _build_sc_gather():
    mesh = plsc.VectorSubcoreMesh(core_axis_name="c", subcore_axis_name="s",
                                  num_cores=NC, num_subcores=NS)

    @functools.partial(
        pl.kernel,
        mesh=mesh,
        out_type=(
            jax.ShapeDtypeStruct((AROWS, LW, ACD), jnp.float32),
            jax.ShapeDtypeStruct((AROWS, LW, ATD), jnp.float32),
            jax.ShapeDtypeStruct((RROWS, LW, RCD), jnp.float32),
            jax.ShapeDtypeStruct((RROWS, LW, RID), jnp.float32),
        ),
        scratch_types=(
            pltpu.VMEM((SUP * LW,), jnp.int32),
            pltpu.VMEM((SUP, LW, ACD), jnp.float32),
            pltpu.VMEM((SUP, LW, ATD), jnp.float32),
            pltpu.SemaphoreType.DMA,
        ),
    )
    def _sc_gather(ac_tab, at_tab, rc_tab, ri_tab, iac, iat, irc, iri,
                   ac_out, at_out, rc_out, ri_out, idx_v, r32_v, r16_v, sem):
        wid = lax.axis_index("s") * NC + lax.axis_index("c")

        def one_batch(tab, idx_hbm, out_hbm, rows_v, row0):
            pltpu.sync_copy(idx_hbm.at[pl.ds(row0 * LW, SUP * LW)], idx_v)
            descs = [
                pltpu.async_copy(tab.at[idx_v.at[pl.ds(j * LW, LW)]],
                                 rows_v.at[j], sem)
                for j in range(SUP)
            ]
            for d in descs:
                d.wait()
            pltpu.sync_copy(rows_v, out_hbm.at[pl.ds(row0, SUP)])

        def atom_body(i, carry):
            row0 = wid * A_ROWS_PER_W + i * SUP
            one_batch(ac_tab, iac, ac_out, r32_v, row0)
            one_batch(at_tab, iat, at_out, r16_v, row0)
            return carry

        lax.fori_loop(0, A_BATCHES, atom_body, 0)

        half = wid // 16
        sub = wid % 16

        def res_body(i, carry):
            row0 = sub * R_ROWS_PER_HALF + i * SUP

            @pl.when(half == 0)
            def _():
                one_batch(rc_tab, irc, rc_out, r32_v, row0)

            @pl.when(half == 1)
            def _():
                one_batch(ri_tab, iri, ri_out, r32_v, row0)

            return carry

        lax.fori_loop(0, R_BATCHES, res_body, 0)

    return _sc_gather


def _mm_body(rel_ref, ac_ref, at_ref, rc_ref, ri_ref,
             w_rel, w_ac, w_at, w_rc, w_ri, b_ref, out_ref):
    acc = jnp.dot(rel_ref[...], w_rel[...], preferred_element_type=jnp.float32)
    acc += jnp.dot(ac_ref[...], w_ac[...], preferred_element_type=jnp.float32)
    acc += jnp.dot(at_ref[...], w_at[...], preferred_element_type=jnp.float32)
    acc += jnp.dot(rc_ref[...], w_rc[...], preferred_element_type=jnp.float32)
    acc += jnp.dot(ri_ref[...], w_ri[...], preferred_element_type=jnp.float32)
    out_ref[...] = acc + b_ref[...]


def _mm(rel, ac, at, rc, ri, W, b):
    w_rel = W[:P * 3]
    w_ac = W[P * 3:P * 3 + P * ACD]
    w_at = W[P * 3 + P * ACD:P * 3 + P * ACD + P * ATD]
    w_rc = W[P * 3 + P * ACD + P * ATD:P * 3 + P * ACD + P * ATD + RCD]
    w_ri = W[P * 3 + P * ACD + P * ATD + RCD:]
    full = lambda s: pl.BlockSpec(s, lambda i: (0, 0))
    return pl.pallas_call(
        _mm_body,
        grid=(NBLK,),
        in_specs=[
            pl.BlockSpec((BLK, P * 3), lambda i: (i, 0)),
            pl.BlockSpec((BLK, P * ACD), lambda i: (i, 0)),
            pl.BlockSpec((BLK, P * ATD), lambda i: (i, 0)),
            pl.BlockSpec((BLK, RCD), lambda i: (i, 0)),
            pl.BlockSpec((BLK, RID), lambda i: (i, 0)),
            full((P * 3, OUT)),
            full((P * ACD, OUT)),
            full((P * ATD, OUT)),
            full((RCD, OUT)),
            full((RID, OUT)),
            pl.BlockSpec((1, OUT), lambda i: (0, 0)),
        ],
        out_specs=pl.BlockSpec((BLK, OUT), lambda i: (i, 0)),
        out_shape=jax.ShapeDtypeStruct((R, OUT), jnp.float32),
    )(rel, ac, at, rc, ri, w_rel, w_ac, w_at, w_rc, w_ri, b.reshape(1, OUT))


def kernel(residue_base_coords, residue_relative_coords, atom_code_index,
           atom_type_index, residue_code_index, residue_sequence_index,
           residue_index_atomwise, atom_code_table, atom_type_table,
           residue_code_table, residue_index_table, W, b):
    i32 = jnp.int32
    pad_a = APAD - A
    pad_r = RPAD - R
    iac = jnp.concatenate([atom_code_index.astype(i32),
                           jnp.zeros((pad_a,), i32)])
    iat = jnp.concatenate([atom_type_index.astype(i32),
                           jnp.zeros((pad_a,), i32)])
    irc = jnp.concatenate([residue_code_index.astype(i32),
                           jnp.zeros((pad_r,), i32)])
    iri = jnp.concatenate([residue_sequence_index.astype(i32),
                           jnp.zeros((pad_r,), i32)])

    ac = jnp.take(atom_code_table, iac, axis=0).reshape(RPAD, P * ACD)
    at = jnp.take(atom_type_table, iat, axis=0).reshape(RPAD, P * ATD)
    rc = jnp.take(residue_code_table, irc, axis=0)
    ri = jnp.take(residue_index_table, iri, axis=0)
    rel = residue_relative_coords.reshape(R, P * 3)

    feats = _mm(rel, ac, at, rc, ri, W, b)
    return (residue_base_coords, feats)


# speedup vs baseline: 2.0888x; 2.0888x over previous
import functools

import jax
import jax.numpy as jnp
from jax import lax
from jax.experimental import pallas as pl
from jax.experimental.pallas import tpu as pltpu
from jax.experimental.pallas import tpu_sc as plsc

R = 100000
P = 14
ACD = 32
ATD = 16
RCD = 32
RID = 32
OUT = 256
A = R * P
AC_V = 38
AT_V = 6
CV = AC_V * AT_V
PV = CV * CV
NPL = P // 2

BLK = 2048
NBLK = 50
RPAD = BLK * NBLK
LW = 128
PROWS = RPAD // LW
AROWS = NPL * PROWS
NC, NS = 2, 16
NW = NC * NS
A_PER_W = AROWS // NW
R_PER_W = PROWS // NW
SUP = 5
A_BATCH = A_PER_W // SUP
R_BATCH = R_PER_W // SUP


@functools.cache
def _build_sc_gather():
    mesh = plsc.VectorSubcoreMesh(core_axis_name="c", subcore_axis_name="s",
                                  num_cores=NC, num_subcores=NS)

    @functools.partial(
        pl.kernel,
        mesh=mesh,
        out_type=(
            jax.ShapeDtypeStruct((AROWS, LW, 128), jnp.float32),
            jax.ShapeDtypeStruct((PROWS, LW, 128), jnp.float32),
            jax.ShapeDtypeStruct((PROWS, LW, 128), jnp.float32),
        ),
        scratch_types=(
            pltpu.VMEM((SUP * LW,), jnp.int32),
            pltpu.VMEM((SUP, LW, 128), jnp.float32),
            pltpu.SemaphoreType.DMA,
        ),
    )
    def _sc_gather(pair_tab, rc_tab, ri_tab, pidx, irc, iri,
                   pair_out, rc_out, ri_out, idx_v, rows_v, sem):
        wid = lax.axis_index("s") * NC + lax.axis_index("c")

        def one_batch(tab, idx_hbm, out_hbm, row0):
            pltpu.sync_copy(idx_hbm.at[pl.ds(row0 * LW, SUP * LW)], idx_v)
            descs = [
                pltpu.async_copy(tab.at[idx_v.at[pl.ds(j * LW, LW)]],
                                 rows_v.at[j], sem)
                for j in range(SUP)
            ]
            for d in descs:
                d.wait()
            pltpu.sync_copy(rows_v, out_hbm.at[pl.ds(row0, SUP)])

        def atom_body(i, carry):
            one_batch(pair_tab, pidx, pair_out, wid * A_PER_W + i * SUP)
            return carry

        lax.fori_loop(0, A_BATCH, atom_body, 0)

        def res_body(i, carry):
            row0 = wid * R_PER_W + i * SUP
            one_batch(rc_tab, irc, rc_out, row0)
            one_batch(ri_tab, iri, ri_out, row0)
            return carry

        lax.fori_loop(0, R_BATCH, res_body, 0)

    return _sc_gather


def _mm_body(rel_ref, pl_ref, rc_ref, ri_ref,
             w_rel, wp, w_rc, w_ri, b_ref, out_ref):
    acc = jnp.dot(rel_ref[...], w_rel[...], preferred_element_type=jnp.float32)
    for s in range(NPL):
        acc += jnp.dot(pl_ref[s], wp[s], preferred_element_type=jnp.float32)
    acc += jnp.dot(rc_ref[...], w_rc[...], preferred_element_type=jnp.float32)
    acc += jnp.dot(ri_ref[...], w_ri[...], preferred_element_type=jnp.float32)
    out_ref[...] = acc + b_ref[...]


def _mm(rel, planes, rc_pl, ri_pl, w_rel, wp, w_rc, w_ri, b):
    full = lambda s: pl.BlockSpec(s, lambda i: (0,) * len(s))
    return pl.pallas_call(
        _mm_body,
        grid=(NBLK,),
        in_specs=[
            pl.BlockSpec((BLK, P * 3), lambda i: (i, 0)),
            pl.BlockSpec((NPL, BLK, 128), lambda i: (0, i, 0)),
            pl.BlockSpec((BLK, 128), lambda i: (i, 0)),
            pl.BlockSpec((BLK, 128), lambda i: (i, 0)),
            full((P * 3, OUT)),
            full((NPL, 128, OUT)),
            full((128, OUT)),
            full((128, OUT)),
            pl.BlockSpec((1, OUT), lambda i: (0, 0)),
        ],
        out_specs=pl.BlockSpec((BLK, OUT), lambda i: (i, 0)),
        out_shape=jax.ShapeDtypeStruct((RPAD, OUT), jnp.float32),
    )(rel, planes, rc_pl, ri_pl, w_rel, wp, w_rc, w_ri, b.reshape(1, OUT))


def kernel(residue_base_coords, residue_relative_coords, atom_code_index,
           atom_type_index, residue_code_index, residue_sequence_index,
           residue_index_atomwise, atom_code_table, atom_type_table,
           residue_code_table, residue_index_table, W, b):
    i32 = jnp.int32
    f32 = jnp.float32

    cidx = (atom_code_index.astype(i32) * AT_V
            + atom_type_index.astype(i32)).reshape(R, P)
    cidx = jnp.pad(cidx, ((0, RPAD - R), (0, 0)))
    pid = cidx[:, 0::2] * CV + cidx[:, 1::2]
    pidx = pid.T.reshape(NPL * RPAD)
    irc = jnp.pad(residue_code_index.astype(i32), (0, RPAD - R))
    iri = jnp.pad(residue_sequence_index.astype(i32), (0, RPAD - R))

    c64 = jnp.concatenate(
        [jnp.repeat(atom_code_table, AT_V, axis=0),
         jnp.tile(atom_type_table, (AC_V, 1)),
         jnp.zeros((CV, 64 - ACD - ATD), f32)], axis=1)
    pair_tab = jnp.concatenate(
        [jnp.broadcast_to(c64[:, None, :], (CV, CV, 64)),
         jnp.broadcast_to(c64[None, :, :], (CV, CV, 64))],
        axis=2).reshape(PV, 128)
    rc_tab = jnp.pad(residue_code_table, ((0, 0), (0, 128 - RCD)))
    ri_tab = jnp.pad(residue_index_table, ((0, 0), (0, 128 - RID)))

    pair_rows, rc_rows, ri_rows = _build_sc_gather()(
        pair_tab, rc_tab, ri_tab, pidx, irc, iri)
    planes = pair_rows.reshape(NPL, RPAD, 128)
    rc_pl = rc_rows.reshape(RPAD, 128)
    ri_pl = ri_rows.reshape(RPAD, 128)

    w_ac = W[P * 3:P * 3 + P * ACD].reshape(P, ACD, OUT)
    w_at = W[P * 3 + P * ACD:P * 3 + P * (ACD + ATD)].reshape(P, ATD, OUT)
    z16 = jnp.zeros((NPL, 64 - ACD - ATD, OUT), f32)
    wp = jnp.concatenate(
        [w_ac[0::2], w_at[0::2], z16, w_ac[1::2], w_at[1::2], z16],
        axis=1)
    k0 = P * (3 + ACD + ATD)
    w_rc = jnp.pad(W[k0:k0 + RCD], ((0, 128 - RCD), (0, 0)))
    w_ri = jnp.pad(W[k0 + RCD:], ((0, 128 - RID), (0, 0)))
    w_rel = W[:P * 3]

    rel = jnp.pad(residue_relative_coords.reshape(R, P * 3),
                  ((0, RPAD - R), (0, 0)))
    feats = _mm(rel, planes, rc_pl, ri_pl, w_rel, wp, w_rc, w_ri, b)
    return (residue_base_coords, feats[:R])
